# baseline (speedup 1.0000x reference)
import jax
import jax.numpy as jnp
from jax.experimental import pallas as pl
from jax.experimental.pallas import tpu as pltpu

MODEL_DIM = 2048
N_EXPERTS = 16
N_TOKENS = 8192
BLK = 2048
GRID = N_TOKENS // BLK


def _gate_block(x_ref, wg_ref, gates_ref, idx_ref, loc_ref,
                laux_ref, cnt_ref, me_ref, triu_ref):
    pid = pl.program_id(0)

    @pl.when(pid == 0)
    def _init():
        cnt_ref[...] = jnp.zeros_like(cnt_ref)
        me_ref[...] = jnp.zeros_like(me_ref)
        ri = jax.lax.broadcasted_iota(jnp.int32, (BLK, BLK), 0)
        ci = jax.lax.broadcasted_iota(jnp.int32, (BLK, BLK), 1)
        triu_ref[...] = (ri < ci).astype(jnp.float32)

    xb = x_ref[...]
    w = wg_ref[...]
    logits = jax.lax.dot_general(
        w, xb, (((1,), (1,)), ((), ())),
        preferred_element_type=jnp.float32)

    maxv = jnp.max(logits, axis=0, keepdims=True)
    ex = jnp.exp(logits - maxv)
    s = jnp.sum(ex, axis=0, keepdims=True)
    gates_ref[...] = jnp.reshape(1.0 / s, (BLK,))

    eidx = jax.lax.broadcasted_iota(jnp.int32, (N_EXPERTS, BLK), 0)
    cand = jnp.where(logits == maxv, eidx, N_EXPERTS)
    idx = jnp.min(cand, axis=0, keepdims=True)
    idx_ref[...] = jnp.reshape(idx, (BLK,))

    mask = (eidx == idx).astype(jnp.float32)

    prev = jax.lax.dot_general(
        mask, triu_ref[...], (((1,), (0,)), ((), ())),
        preferred_element_type=jnp.float32)

    carried = cnt_ref[...]
    loc = jnp.sum((prev + carried) * mask, axis=0, keepdims=True)
    loc_ref[...] = jnp.reshape(loc, (BLK,)).astype(jnp.int32)

    cnt_ref[...] = carried + jnp.sum(mask, axis=1, keepdims=True)
    me_ref[...] = me_ref[...] + jnp.sum(ex / s, axis=1, keepdims=True)

    @pl.when(pid == GRID - 1)
    def _fini():
        me = me_ref[...]
        ce = cnt_ref[...]
        val = jnp.sum(me * ce) * (N_EXPERTS / (N_TOKENS * N_TOKENS))
        laux_ref[...] = jnp.full((1, 1), val, dtype=jnp.float32)


def kernel(input, wg):
    out_shapes = (
        jax.ShapeDtypeStruct((N_TOKENS,), jnp.float32),
        jax.ShapeDtypeStruct((N_TOKENS,), jnp.int32),
        jax.ShapeDtypeStruct((N_TOKENS,), jnp.int32),
        jax.ShapeDtypeStruct((1, 1), jnp.float32),
    )
    gates1_s, idx, loc, laux = pl.pallas_call(
        _gate_block,
        grid=(GRID,),
        in_specs=[
            pl.BlockSpec((BLK, MODEL_DIM), lambda i: (i, 0)),
            pl.BlockSpec((N_EXPERTS, MODEL_DIM), lambda i: (0, 0)),
        ],
        out_specs=(
            pl.BlockSpec((BLK,), lambda i: (i,)),
            pl.BlockSpec((BLK,), lambda i: (i,)),
            pl.BlockSpec((BLK,), lambda i: (i,)),
            pl.BlockSpec((1, 1), lambda i: (0, 0)),
        ),
        out_shape=out_shapes,
        scratch_shapes=[
            pltpu.VMEM((N_EXPERTS, 1), jnp.float32),
            pltpu.VMEM((N_EXPERTS, 1), jnp.float32),
            pltpu.VMEM((BLK, BLK), jnp.float32),
        ],
    )(input, wg)
    return (laux[0, 0], gates1_s, idx, loc)


# speedup vs baseline: 1.0594x; 1.0594x over previous
import jax
import jax.numpy as jnp
from jax.experimental import pallas as pl
from jax.experimental.pallas import tpu as pltpu

MODEL_DIM = 2048
N_EXPERTS = 16
N_TOKENS = 8192
BLK = 1024
KHALF = MODEL_DIM // 2
GRID = N_TOKENS // BLK


def _gate_block(xa_ref, xb_ref, wg_ref, gates_ref, idx_ref, loc_ref,
                laux_ref, cnt_ref, me_ref, triu_ref):
    pid = pl.program_id(0)

    @pl.when(pid == 0)
    def _init():
        cnt_ref[...] = jnp.zeros_like(cnt_ref)
        me_ref[...] = jnp.zeros_like(me_ref)
        ri = jax.lax.broadcasted_iota(jnp.int32, (BLK, BLK), 0)
        ci = jax.lax.broadcasted_iota(jnp.int32, (BLK, BLK), 1)
        triu_ref[...] = (ri < ci).astype(jnp.float32)

    w = wg_ref[...]
    logits = jax.lax.dot_general(
        w[:, :KHALF], xa_ref[...], (((1,), (1,)), ((), ())),
        preferred_element_type=jnp.float32)
    logits = logits + jax.lax.dot_general(
        w[:, KHALF:], xb_ref[...], (((1,), (1,)), ((), ())),
        preferred_element_type=jnp.float32)

    maxv = jnp.max(logits, axis=0, keepdims=True)
    ex = jnp.exp(logits - maxv)
    s = jnp.sum(ex, axis=0, keepdims=True)
    gates_ref[...] = jnp.reshape(1.0 / s, (BLK,))

    eidx = jax.lax.broadcasted_iota(jnp.int32, (N_EXPERTS, BLK), 0)
    cand = jnp.where(logits == maxv, eidx, N_EXPERTS)
    idx = jnp.min(cand, axis=0, keepdims=True)
    idx_ref[...] = jnp.reshape(idx, (BLK,))

    mask = (eidx == idx).astype(jnp.float32)

    prev = jax.lax.dot_general(
        mask, triu_ref[...], (((1,), (0,)), ((), ())),
        preferred_element_type=jnp.float32)

    carried = cnt_ref[...]
    loc = jnp.sum((prev + carried) * mask, axis=0, keepdims=True)
    loc_ref[...] = jnp.reshape(loc, (BLK,)).astype(jnp.int32)

    cnt_ref[...] = carried + jnp.sum(mask, axis=1, keepdims=True)
    me_ref[...] = me_ref[...] + jnp.sum(ex / s, axis=1, keepdims=True)

    @pl.when(pid == GRID - 1)
    def _fini():
        me = me_ref[...]
        ce = cnt_ref[...]
        val = jnp.sum(me * ce) * (N_EXPERTS / (N_TOKENS * N_TOKENS))
        laux_ref[...] = jnp.full((1, 1), val, dtype=jnp.float32)


def kernel(input, wg):
    out_shapes = (
        jax.ShapeDtypeStruct((N_TOKENS,), jnp.float32),
        jax.ShapeDtypeStruct((N_TOKENS,), jnp.int32),
        jax.ShapeDtypeStruct((N_TOKENS,), jnp.int32),
        jax.ShapeDtypeStruct((1, 1), jnp.float32),
    )
    gates1_s, idx, loc, laux = pl.pallas_call(
        _gate_block,
        grid=(GRID,),
        in_specs=[
            pl.BlockSpec((BLK, KHALF), lambda i: (i, 0)),
            pl.BlockSpec((BLK, KHALF), lambda i: (i, 1)),
            pl.BlockSpec((N_EXPERTS, MODEL_DIM), lambda i: (0, 0)),
        ],
        out_specs=(
            pl.BlockSpec((BLK,), lambda i: (i,)),
            pl.BlockSpec((BLK,), lambda i: (i,)),
            pl.BlockSpec((BLK,), lambda i: (i,)),
            pl.BlockSpec((1, 1), lambda i: (0, 0)),
        ),
        out_shape=out_shapes,
        scratch_shapes=[
            pltpu.VMEM((N_EXPERTS, 1), jnp.float32),
            pltpu.VMEM((N_EXPERTS, 1), jnp.float32),
            pltpu.VMEM((BLK, BLK), jnp.float32),
        ],
    )(input, input, wg)
    return (laux[0, 0], gates1_s, idx, loc)
